# Initial kernel scaffold; baseline (speedup 1.0000x reference)
import jax
import jax.numpy as jnp
from jax.experimental import pallas as pl


def kernel(x, edge_index, forward_level, gate, rc_pair_index, Wc_and_strc, bc_and_strc, Wih_and_strc, Whh_and_strc, bih_and_strc, bhh_and_strc, Wc_and_func, bc_and_func, Wih_and_func, Whh_and_func, bih_and_func, bhh_and_func, Wc_or_strc, bc_or_strc, Wih_or_strc, Whh_or_strc, bih_or_strc, bhh_or_strc, Wc_or_func, bc_or_func, Wih_or_func, Whh_or_func, bih_or_func, bhh_or_func, W_prob, b_prob, W_rc, b_rc):
    raise NotImplementedError("write your pallas kernel here")


# speedup vs baseline: 519.7709x; 519.7709x over previous
import functools

import jax
import jax.numpy as jnp
from jax import lax
from jax.experimental import pallas as pl
from jax.experimental.pallas import tpu as pltpu
from jax.experimental.pallas import tpu_sc as plsc

N = 10000
E = 320000
H = 128
NPAD = 10240
ROWS = NPAD // 128
P = 4096
NC = 2
NS = 16
NW = NC * NS
EPT = E // NW
EG = EPT // 16
DUMMY = N

_mesh = plsc.VectorSubcoreMesh(core_axis_name="c", subcore_axis_name="s")


def _wid():
    return lax.axis_index("s") * NC + lax.axis_index("c")


def _iota16():
    return lax.iota(jnp.int32, 16)


@functools.partial(
    pl.kernel,
    out_type=(
        jax.ShapeDtypeStruct((NC, ROWS, 128), jnp.float32),
        jax.ShapeDtypeStruct((NC, ROWS, 128), jnp.float32),
        jax.ShapeDtypeStruct((NW, ROWS, 128), jnp.int32),
        jax.ShapeDtypeStruct((NW, ROWS, 128), jnp.int32),
        jax.ShapeDtypeStruct((NW, 16), jnp.int32),
    ),
    mesh=_mesh,
    compiler_params=pltpu.CompilerParams(needs_layout_passes=False),
    scratch_types=[
        pltpu.VMEM((N,), jnp.int32),
        pltpu.VMEM((N,), jnp.int32),
        pltpu.VMEM((N,), jnp.int32),
        pltpu.VMEM((EPT,), jnp.int32),
        pltpu.VMEM((EPT,), jnp.int32),
        pltpu.VMEM((ROWS, 128), jnp.float32),
        pltpu.VMEM((ROWS, 128), jnp.float32),
        pltpu.VMEM((ROWS, 128), jnp.int32),
        pltpu.VMEM((ROWS, 128), jnp.int32),
        pltpu.VMEM((ROWS, 128), jnp.float32),
        pltpu.VMEM((ROWS,), jnp.int32),
        pltpu.VMEM((16,), jnp.int32),
        pltpu.VMEM_SHARED((ROWS, 128), jnp.float32),
        pltpu.VMEM_SHARED((ROWS, 128), jnp.float32),
    ],
)
def _sc_edges(lv_hbm, gt_hbm, src_hbm, dst_hbm,
              dego_hbm, degi_hbm, csrc_hbm, cdst_hbm, counts_hbm,
              lv_v, gt_v, key_v, es_v, ed_v, dego_v, degi_v, csrc_v, cdst_v,
              zbuf, idxv, stage16, dego_sh, degi_sh):
    c = lax.axis_index("c")
    s = lax.axis_index("s")
    wid = _wid()

    pltpu.sync_copy(lv_hbm, lv_v)
    pltpu.sync_copy(gt_hbm, gt_v)
    pltpu.sync_copy(src_hbm.at[pl.ds(wid * EPT, EPT)], es_v)
    pltpu.sync_copy(dst_hbm.at[pl.ds(wid * EPT, EPT)], ed_v)

    def key_body(i, _):
        lv16 = lv_v[pl.ds(i * 16, 16)]
        gt16 = gt_v[pl.ds(i * 16, 16)]
        valid = ((lv16 >= 1) & (lv16 <= 7)) & ((gt16 == 1) | (gt16 == 2))
        key_v[pl.ds(i * 16, 16)] = jnp.where(valid, lv16 * 4 + gt16,
                                             jnp.full((16,), -1, jnp.int32))
        return 0
    lax.fori_loop(0, N // 16, key_body, 0)

    zf = jnp.zeros((16,), jnp.float32)
    zi = jnp.zeros((16,), jnp.int32)
    dumv = jnp.full((16,), DUMMY, jnp.int32)

    def zero_body(j, _):
        r = j >> 3
        k = j & 7
        dego_v[r, pl.ds(k * 16, 16)] = zf
        degi_v[r, pl.ds(k * 16, 16)] = zf
        zbuf[r, pl.ds(k * 16, 16)] = zf
        csrc_v[r, pl.ds(k * 16, 16)] = zi
        cdst_v[r, pl.ds(k * 16, 16)] = dumv
        return 0
    lax.fori_loop(0, ROWS * 8, zero_body, 0)

    ones_f = jnp.ones((16,), jnp.float32)

    def edge_body(i, cnt):
        s16 = es_v[pl.ds(i * 16, 16)]
        d16 = ed_v[pl.ds(i * 16, 16)]
        ks = plsc.load_gather(key_v, [s16])
        kd = plsc.load_gather(key_v, [d16])
        m = (ks == kd) & (ks >= 0)
        plsc.addupdate_scatter(dego_v, [s16 >> 7, s16 & 127], ones_f, mask=m)
        plsc.addupdate_scatter(degi_v, [d16 >> 7, d16 & 127], ones_f, mask=m)
        mi = m.astype(jnp.int32)
        pos = cnt + jnp.cumsum(mi) - 1
        plsc.store_scatter(csrc_v, [pos >> 7, pos & 127], s16, mask=m)
        plsc.store_scatter(cdst_v, [pos >> 7, pos & 127], d16, mask=m)
        return cnt + jnp.sum(mi)
    cnt = lax.fori_loop(0, EG, edge_body, jnp.int32(0))

    pltpu.sync_copy(csrc_v, csrc_hbm.at[wid])
    pltpu.sync_copy(cdst_v, cdst_hbm.at[wid])
    stage16[...] = jnp.full((16,), cnt, jnp.int32)
    pltpu.sync_copy(stage16, counts_hbm.at[wid])

    def idx_body(k, _):
        idxv[pl.ds(k * 16, 16)] = k * 16 + _iota16()
        return 0
    lax.fori_loop(0, ROWS // 16, idx_body, 0)

    @pl.when(s == 0)
    def _():
        pltpu.sync_copy(zbuf, dego_sh)
        pltpu.sync_copy(zbuf, degi_sh)
    plsc.subcore_barrier()
    pltpu.sync_copy(dego_v, dego_sh.at[idxv], add=True)
    pltpu.sync_copy(degi_v, degi_sh.at[idxv], add=True)
    plsc.subcore_barrier()

    @pl.when(s == 0)
    def _():
        pltpu.sync_copy(dego_sh, dego_hbm.at[c])
        pltpu.sync_copy(degi_sh, degi_hbm.at[c])


def _nrsqrt(x):
    i = plsc.bitcast(x, jnp.int32)
    y = plsc.bitcast(jnp.int32(0x5F3759DF) - (i >> 1), jnp.float32)
    for _ in range(3):
        y = y * (1.5 - 0.5 * x * y * y)
    return y


CH = 64


@functools.partial(
    pl.kernel,
    out_type=(
        jax.ShapeDtypeStruct((NC, NPAD, 128), jnp.float32),
        jax.ShapeDtypeStruct((NC, ROWS, 128), jnp.float32),
    ),
    mesh=_mesh,
    compiler_params=pltpu.CompilerParams(needs_layout_passes=False),
    scratch_types=[
        pltpu.VMEM((NPAD,), jnp.float32),
        pltpu.VMEM((NPAD,), jnp.float32),
        pltpu.VMEM((CH,), jnp.int32),
        pltpu.VMEM((CH,), jnp.int32),
        pltpu.VMEM((CH, 128), jnp.float32),
        pltpu.VMEM((ROWS, 128), jnp.float32),
        pltpu.VMEM((CH,), jnp.float32),
        pltpu.VMEM((ROWS,), jnp.int32),
        pltpu.VMEM((16,), jnp.int32),
        pltpu.SemaphoreType.DMA,
        pltpu.VMEM_SHARED((NPAD, 128), jnp.float32),
        pltpu.VMEM_SHARED((ROWS, 128), jnp.float32),
    ],
)
def _sc_gather(x_hbm, dego_p_hbm, csrc_hbm, cdst_hbm, counts_hbm,
               a_hbm, s_hbm,
               dego_v, tmp_v, csrcbuf, cdstbuf, rows_v, S2d, ns_chunk, idxv,
               cnt16, sem, A_sh, S_sh):
    c = lax.axis_index("c")
    s = lax.axis_index("s")
    wid = _wid()

    zf = jnp.zeros((16,), jnp.float32)

    def zero_body(j, _):
        r = j >> 3
        k = j & 7
        rows_v[r, pl.ds(k * 16, 16)] = zf
        return 0
    lax.fori_loop(0, CH * 8, zero_body, 0)

    def zero_s_body(j, _):
        r = j >> 3
        k = j & 7
        S2d[r, pl.ds(k * 16, 16)] = zf
        return 0
    lax.fori_loop(0, ROWS * 8, zero_s_body, 0)

    for q in range(NPAD // CH // NS):
        pltpu.sync_copy(rows_v, A_sh.at[pl.ds((s * 10 + q) * CH, CH)])

    @pl.when(s == 0)
    def _():
        pltpu.sync_copy(S2d, S_sh)

    pltpu.sync_copy(dego_p_hbm.at[0], dego_v)
    pltpu.sync_copy(dego_p_hbm.at[1], tmp_v)

    def merge_body(j, _):
        sl = pl.ds(j * 16, 16)
        dego_v[sl] = dego_v[sl] + tmp_v[sl]
        return 0
    lax.fori_loop(0, NPAD // 16, merge_body, 0)

    pltpu.sync_copy(counts_hbm.at[wid], cnt16)
    cnt = cnt16[pl.ds(0, 16)][0]
    nchunks = (cnt + CH - 1) >> 6

    plsc.subcore_barrier()

    def chunk_body(j, _):
        pltpu.sync_copy(csrc_hbm.at[wid, pl.ds(j * CH, CH)], csrcbuf)
        pltpu.sync_copy(cdst_hbm.at[wid, pl.ds(j * CH, CH)], cdstbuf)
        pltpu.async_copy(x_hbm.at[csrcbuf], rows_v, sem).wait()
        for k in range(CH // 16):
            s16 = csrcbuf[pl.ds(k * 16, 16)]
            d16 = cdstbuf[pl.ds(k * 16, 16)]
            dg = plsc.load_gather(dego_v, [s16])
            ns16 = _nrsqrt(jnp.maximum(dg, 1.0))
            plsc.addupdate_scatter(S2d, [d16 >> 7, d16 & 127], ns16)
            ns_chunk[pl.ds(k * 16, 16)] = ns16

        def rbody(r, _):
            cf = plsc.load_gather(ns_chunk, [jnp.full((16,), r, jnp.int32)])
            for h2 in range(8):
                sl = pl.ds(h2 * 16, 16)
                rows_v[r, sl] = rows_v[r, sl] * cf
            return 0
        lax.fori_loop(0, CH, rbody, 0)
        pltpu.sync_copy(rows_v, A_sh.at[cdstbuf], add=True)
        return 0
    lax.fori_loop(0, nchunks, chunk_body, 0)

    def idx_body(k, _):
        idxv[pl.ds(k * 16, 16)] = k * 16 + _iota16()
        return 0
    lax.fori_loop(0, ROWS // 16, idx_body, 0)
    pltpu.sync_copy(S2d, S_sh.at[idxv], add=True)
    plsc.subcore_barrier()

    @pl.when(s == 0)
    def _():
        pltpu.sync_copy(A_sh, a_hbm.at[c])
        pltpu.sync_copy(S_sh, s_hbm.at[c])


BR = 1024


def _tc_body(a0, a1, s0, s1, di0, di1, gt, lv,
             wcs_a, wcf_a, wih_sa, whh_sa, wih_fa, whh_fa,
             wcs_o, wcf_o, wih_so, whh_so, wih_fo, whh_fo,
             bcs_a, bcf_a, bih_sa, bhh_sa, bih_fa, bhh_fa,
             bcs_o, bcf_o, bih_so, bhh_so, bih_fo, bhh_fo,
             wprob, wrc, bprob, brc,
             hs_o, hf_o, prob_o, q1_o, q2_o):
    f32 = jnp.float32
    A = a0[...] + a1[...]
    di = di0[...] + di1[...]
    ni = lax.rsqrt(jnp.maximum(di, 1.0))
    Ai = A * ni
    S = (s0[...] + s1[...]) * ni

    def gru1(msg, Wih, Whh, bih, bhh):
        gi = lax.dot_general(msg, Wih[...], (((1,), (1,)), ((), ())),
                             preferred_element_type=f32)
        gh = jnp.sum(Whh[...], axis=1) + bhh[...]
        r = jax.nn.sigmoid(gi[:, :H] + gh[None, :H])
        z = jax.nn.sigmoid(gi[:, H:2 * H] + gh[None, H:2 * H])
        n_ = jnp.tanh(gi[:, 2 * H:] + r * gh[None, 2 * H:])
        return (1.0 - z) * n_ + z

    outs = {}
    for nm, (wcs, wcf, wih_s, whh_s, wih_f, whh_f,
             bcs, bcf, bih_s, bhh_s, bih_f, bhh_f) in {
            'and': (wcs_a, wcf_a, wih_sa, whh_sa, wih_fa, whh_fa,
                    bcs_a, bcf_a, bih_sa, bhh_sa, bih_fa, bhh_fa),
            'or': (wcs_o, wcf_o, wih_so, whh_so, wih_fo, whh_fo,
                   bcs_o, bcf_o, bih_so, bhh_so, bih_fo, bhh_fo)}.items():
        msg_s = jnp.maximum(
            lax.dot_general(Ai, wcs[...], (((1,), (0,)), ((), ())),
                            preferred_element_type=f32) + bcs[...][None, :],
            0.0)
        wf = wcf[...]
        wfb = jnp.sum(wf[H:, :], axis=0)
        msg_f = jnp.maximum(
            lax.dot_general(Ai, wf[:H, :], (((1,), (0,)), ((), ())),
                            preferred_element_type=f32)
            + S * wfb[None, :] + bcf[...][None, :], 0.0)
        outs['hs_' + nm] = gru1(msg_s, wih_s, whh_s, bih_s[...], bhh_s[...])
        outs['hf_' + nm] = gru1(msg_f, wih_f, whh_f, bih_f[...], bhh_f[...])

    gti = gt[...]
    lvi = lv[...]
    valid = ((lvi >= 1) & (lvi <= 7)) & ((gti == 1) | (gti == 2))
    is_and = gti == 1
    hs = jnp.where(valid, jnp.where(is_and, outs['hs_and'], outs['hs_or']), 1.0)
    hf = jnp.where(valid, jnp.where(is_and, outs['hf_and'], outs['hf_or']), 1.0)
    hs_o[...] = hs
    hf_o[...] = hf
    ph = lax.dot_general(hf, wprob[...], (((1,), (0,)), ((), ())),
                         preferred_element_type=f32)
    prob_o[...] = jax.nn.sigmoid(ph[:, 0:1] + bprob[0, 0])
    qh = lax.dot_general(hs, wrc[...], (((1,), (0,)), ((), ())),
                         preferred_element_type=f32)
    q1_o[...] = qh[:, 0:1] + brc[0, 0]
    q2_o[...] = qh[:, 1:2]


def _tc_dense(a0, a1, s0, s1, di0, di1, gt2, lv2, weights):
    nblk = NPAD // BR
    row_spec = pl.BlockSpec((BR, 128), lambda i: (i, 0))
    col_spec = pl.BlockSpec((BR, 1), lambda i: (i, 0))

    def full(arr):
        return pl.BlockSpec(arr.shape, lambda i: tuple(0 for _ in arr.shape))

    in_specs = ([row_spec, row_spec, col_spec, col_spec, col_spec, col_spec,
                 col_spec, col_spec] + [full(w) for w in weights])
    out_specs = (row_spec, row_spec, col_spec, col_spec, col_spec)
    out_shape = (
        jax.ShapeDtypeStruct((NPAD, 128), jnp.float32),
        jax.ShapeDtypeStruct((NPAD, 128), jnp.float32),
        jax.ShapeDtypeStruct((NPAD, 1), jnp.float32),
        jax.ShapeDtypeStruct((NPAD, 1), jnp.float32),
        jax.ShapeDtypeStruct((NPAD, 1), jnp.float32),
    )
    return pl.pallas_call(
        _tc_body,
        grid=(nblk,),
        in_specs=in_specs,
        out_specs=out_specs,
        out_shape=out_shape,
        compiler_params=pltpu.CompilerParams(
            dimension_semantics=("arbitrary",)),
    )(a0, a1, s0, s1, di0, di1, gt2, lv2, *weights)


@functools.partial(
    pl.kernel,
    out_type=jax.ShapeDtypeStruct((NW, 128), jnp.float32),
    mesh=_mesh,
    compiler_params=pltpu.CompilerParams(needs_layout_passes=False),
    scratch_types=[
        pltpu.VMEM((NPAD,), jnp.float32),
        pltpu.VMEM((NPAD,), jnp.float32),
        pltpu.VMEM((128,), jnp.int32),
        pltpu.VMEM((128,), jnp.int32),
        pltpu.VMEM((128,), jnp.float32),
    ],
)
def _sc_rc(q1_hbm, q2_hbm, rc0_hbm, rc1_hbm, out_hbm, q1_v, q2_v, r0, r1, o):
    wid = _wid()
    pltpu.sync_copy(q1_hbm, q1_v)
    pltpu.sync_copy(q2_hbm, q2_v)
    pltpu.sync_copy(rc0_hbm.at[pl.ds(wid * 128, 128)], r0)
    pltpu.sync_copy(rc1_hbm.at[pl.ds(wid * 128, 128)], r1)
    for k in range(8):
        sl = pl.ds(k * 16, 16)
        g = (plsc.load_gather(q1_v, [r0[sl]])
             + plsc.load_gather(q2_v, [r1[sl]]))
        o[sl] = 1.0 / (1.0 + jnp.exp(-g))
    pltpu.sync_copy(o, out_hbm.at[wid])


def kernel(x, edge_index, forward_level, gate, rc_pair_index,
           Wc_and_strc, bc_and_strc, Wih_and_strc, Whh_and_strc,
           bih_and_strc, bhh_and_strc,
           Wc_and_func, bc_and_func, Wih_and_func, Whh_and_func,
           bih_and_func, bhh_and_func,
           Wc_or_strc, bc_or_strc, Wih_or_strc, Whh_or_strc,
           bih_or_strc, bhh_or_strc,
           Wc_or_func, bc_or_func, Wih_or_func, Whh_or_func,
           bih_or_func, bhh_or_func,
           W_prob, b_prob, W_rc, b_rc):
    src = edge_index[0]
    dst = edge_index[1]

    dego_p, degi_p, csrc, cdst, counts = _sc_edges(
        forward_level, gate, src, dst)

    a_p, s_p = _sc_gather(
        x, dego_p.reshape(NC, NPAD), csrc.reshape(NW, NPAD),
        cdst.reshape(NW, NPAD), counts)

    a0 = a_p[0]
    a1 = a_p[1]
    s0 = s_p[0].reshape(NPAD, 1)
    s1 = s_p[1].reshape(NPAD, 1)
    di0 = degi_p[0].reshape(NPAD, 1)
    di1 = degi_p[1].reshape(NPAD, 1)
    gt2 = jnp.pad(gate, (0, NPAD - N)).reshape(NPAD, 1)
    lv2 = jnp.pad(forward_level, (0, NPAD - N)).reshape(NPAD, 1)

    weights = [
        Wc_and_strc, Wc_and_func, Wih_and_strc, Whh_and_strc,
        Wih_and_func, Whh_and_func,
        Wc_or_strc, Wc_or_func, Wih_or_strc, Whh_or_strc,
        Wih_or_func, Whh_or_func,
        bc_and_strc, bc_and_func, bih_and_strc, bhh_and_strc,
        bih_and_func, bhh_and_func,
        bc_or_strc, bc_or_func, bih_or_strc, bhh_or_strc,
        bih_or_func, bhh_or_func,
        jnp.pad(W_prob.T, ((0, 0), (0, 127))),
        jnp.pad(W_rc.reshape(2, H).T, ((0, 0), (0, 126))),
        b_prob.reshape(1, 1), b_rc.reshape(1, 1),
    ]
    hs_pad, hf_pad, prob_pad, q1, q2 = _tc_dense(
        a0, a1, s0, s1, di0, di1, gt2, lv2, weights)

    isrc = _sc_rc(q1.reshape(NPAD), q2.reshape(NPAD),
                  rc_pair_index[0], rc_pair_index[1])

    return (hs_pad[:N], hf_pad[:N], prob_pad[:N], isrc.reshape(P, 1))
